# baseline (speedup 1.0000x reference)
import functools

import jax
import jax.numpy as jnp
from jax import lax
from jax.experimental import pallas as pl
from jax.experimental.pallas import tpu as pltpu
from jax.experimental.pallas import tpu_sc as plsc

B, N = 64, 32768
D, K = 8192, 8
NC, NS = 2, 16
BH = B // NC
DPT = D // NS
GD = 64
GR = GD * K
NJ = GR // 128
NG = DPT // GD
CH = 16
RPT = N // NS
RB = 128
NBLK = RPT // RB


def _body(x, det, out, x2, stat, *rest):
    idx_flat, rest = rest[:3 * 2 * NJ], rest[3 * 2 * NJ:]
    idxr = [list(idx_flat[0:NJ]), list(idx_flat[NJ:2 * NJ])]
    idxg = [list(idx_flat[2 * NJ:3 * NJ]), list(idx_flat[3 * NJ:4 * NJ])]
    idxs = [list(idx_flat[4 * NJ:5 * NJ]), list(idx_flat[5 * NJ:6 * NJ])]
    (vals0, vals1, flags0, flags1, sb0, sb1, vb0, vb1, ob0, ob1,
     semg0, semg1, sems0, sems1, semr0, semr1, semw0, semw1, semz) = rest
    vals = [vals0, vals1]
    flags = [flags0, flags1]
    sbuf = [sb0, sb1]
    vbuf = [vb0, vb1]
    obuf = [ob0, ob1]
    semg = [semg0, semg1]
    sems = [sems0, sems1]
    semr = [semr0, semr1]
    semw = [semw0, semw1]

    c = lax.axis_index("c")
    s = lax.axis_index("s")
    cn = c * N
    n0 = s * RPT
    iota = lax.iota(jnp.int32, 16)
    zero16 = jnp.zeros((16,), jnp.int32)

    def zrow(r, cc):
        flags0[r, :] = jnp.zeros((BH,), jnp.int16)
        return cc

    lax.fori_loop(0, GR, zrow, 0)
    for j in range(RPT // GR):
        pltpu.sync_copy(flags0, stat.at[pl.ds(n0 + j * GR, GR)])

    def t_blk(j, cc):
        nb = n0 + j * RB
        pltpu.sync_copy(x.at[pl.ds(c * BH, BH), pl.ds(nb, RB)], ob0)
        for b in range(BH):
            col = zero16 + b
            for t in range(RB // 16):
                plsc.store_scatter(vb0, [iota + t * 16, col],
                                   ob0[b, pl.ds(t * 16, 16)])
        pltpu.sync_copy(vb0, x2.at[pl.ds(cn + nb, RB)])
        return cc

    lax.fori_loop(0, NBLK, t_blk, 0)
    plsc.subcore_barrier()

    def load_idx(p, g):
        off = s * (DPT * K) + g * GR
        for j in range(NJ):
            pltpu.sync_copy(det.at[pl.ds(off + j * 128, 128)], idxr[p][j])
        for j in range(NJ):
            for t in range(8):
                sl = pl.ds(t * 16, 16)
                idxg[p][j][sl] = idxr[p][j][sl] + cn

    def issue_gather(p):
        for j in range(NJ):
            pltpu.async_copy(x2.at[idxg[p][j]],
                             vals[p].at[pl.ds(j * 128, 128)], semg[p])

    def wait_gather(p):
        for j in range(NJ):
            pltpu.make_async_copy(x2.at[idxg[p][j]],
                                  vals[p].at[pl.ds(j * 128, 128)],
                                  semg[p]).wait()

    def issue_scatter(p):
        for j in range(NJ):
            for t in range(8):
                sl = pl.ds(t * 16, 16)
                idxs[p][j][sl] = idxr[p][j][sl]
        for j in range(NJ):
            pltpu.async_copy(flags[p].at[pl.ds(j * 128, 128)],
                             stat.at[idxs[p][j]], sems[p], add=True)

    def wait_scatter(p):
        for j in range(NJ):
            pltpu.make_async_copy(flags[p].at[pl.ds(j * 128, 128)],
                                  stat.at[idxs[p][j]], sems[p]).wait()

    def compute(p):
        vp = vals[p]
        fp = flags[p]

        def chunk_body(q, cc):
            cb = q * (CH * K)
            for g in range(CH):
                r0 = g * K
                packed = []
                for h in (0, 1):
                    sl = pl.ds(h * 16, 16)
                    v = [vp[cb + r0 + k, sl] for k in range(K)]
                    m = v[0]
                    for k in range(1, K):
                        m = jnp.maximum(m, v[k])
                    eq = v[0] == m
                    wins = [eq]
                    seen = eq
                    for k in range(1, K):
                        eq = v[k] == m
                        wins.append(eq & ~seen)
                        seen = seen | eq
                    packed.append([jnp.where(w, 0, 1).astype(jnp.int32)
                                   for w in wins])
                for k in range(K):
                    both = packed[0][k] | lax.shift_left(packed[1][k], 16)
                    fp[cb + r0 + k, :] = plsc.bitcast(both, jnp.int16)
            return cc

        lax.fori_loop(0, GD // CH, chunk_body, 0)

    load_idx(0, 0)
    issue_gather(0)

    def pair(i, cc):
        ga = 2 * i
        load_idx(1, ga + 1)
        issue_gather(1)
        wait_gather(0)

        @pl.when(i > 0)
        def _():
            wait_scatter(0)

        compute(0)
        issue_scatter(0)

        @pl.when(i < NG // 2 - 1)
        def _():
            load_idx(0, ga + 2)
            issue_gather(0)

        wait_gather(1)

        @pl.when(i > 0)
        def _():
            wait_scatter(1)

        compute(1)
        issue_scatter(1)
        return cc

    lax.fori_loop(0, NG // 2, pair, 0)
    wait_scatter(0)
    wait_scatter(1)
    plsc.subcore_barrier()

    def o_blk(j, cc):
        r0 = n0 + j * RB
        pltpu.sync_copy(stat.at[pl.ds(r0, RB)], sb0)
        pltpu.sync_copy(x2.at[pl.ds(cn + r0, RB)], vb0)
        for r in range(RB):
            w = plsc.bitcast(sb0[r, :], jnp.int32)
            a = w & 0xFFFF
            bm = lax.shift_right_logical(w, 16)
            o0 = jnp.where(a == 0, vb0[r, pl.ds(0, 16)], 0.0)
            o1 = jnp.where(bm == 0, vb0[r, pl.ds(16, 16)], 0.0)
            col = zero16 + r
            plsc.store_scatter(ob0, [iota, col], o0)
            plsc.store_scatter(ob0, [iota + 16, col], o1)
        pltpu.sync_copy(ob0, out.at[pl.ds(c * BH, BH), pl.ds(r0, RB)])
        return cc

    lax.fori_loop(0, NBLK, o_blk, 0)


_sc_call = functools.partial(
    pl.kernel,
    out_type=jax.ShapeDtypeStruct((B, N), jnp.float32),
    mesh=plsc.VectorSubcoreMesh(core_axis_name="c", subcore_axis_name="s"),
    compiler_params=pltpu.CompilerParams(
        needs_layout_passes=False, use_tc_tiling_on_sc=False),
    scratch_types=(
        [pltpu.HBM((NC * N, BH), jnp.float32),
         pltpu.VMEM_SHARED((N, BH), jnp.int16)]
        + [pltpu.VMEM((128,), jnp.int32)] * (3 * 2 * NJ)
        + [pltpu.VMEM((GR, BH), jnp.float32),
           pltpu.VMEM((GR, BH), jnp.float32),
           pltpu.VMEM((GR, BH), jnp.int16),
           pltpu.VMEM((GR, BH), jnp.int16),
           pltpu.VMEM((RB, BH), jnp.int16),
           pltpu.VMEM((RB, BH), jnp.int16),
           pltpu.VMEM((RB, BH), jnp.float32),
           pltpu.VMEM((RB, BH), jnp.float32),
           pltpu.VMEM((BH, RB), jnp.float32),
           pltpu.VMEM((BH, RB), jnp.float32),
           pltpu.SemaphoreType.DMA,
           pltpu.SemaphoreType.DMA,
           pltpu.SemaphoreType.DMA,
           pltpu.SemaphoreType.DMA,
           pltpu.SemaphoreType.DMA,
           pltpu.SemaphoreType.DMA,
           pltpu.SemaphoreType.DMA,
           pltpu.SemaphoreType.DMA,
           pltpu.SemaphoreType.DMA]
    ),
)(_body)


@jax.jit
def kernel(x, detectors):
    return _sc_call(x, detectors.reshape(-1))


# speedup vs baseline: 1.3875x; 1.3875x over previous
import functools

import jax
import jax.numpy as jnp
from jax import lax
from jax.experimental import pallas as pl
from jax.experimental.pallas import tpu as pltpu
from jax.experimental.pallas import tpu_sc as plsc

B, N = 64, 32768
D, K = 8192, 8
NC, NS = 2, 16
BH = B // NC
DPT = D // NS
GD = 64
GR = GD * K
NJ = GR // 128
NG = DPT // GD
CH = 16
RPT = N // NS
RB = 128
NBLK = RPT // RB


def _body(x2, det, out, stat, *rest):
    idx_flat, rest = rest[:3 * 2 * NJ], rest[3 * 2 * NJ:]
    idxr = [list(idx_flat[0:NJ]), list(idx_flat[NJ:2 * NJ])]
    idxg = [list(idx_flat[2 * NJ:3 * NJ]), list(idx_flat[3 * NJ:4 * NJ])]
    idxs = [list(idx_flat[4 * NJ:5 * NJ]), list(idx_flat[5 * NJ:6 * NJ])]
    (vals0, vals1, flags0, flags1,
     semg0, semg1, sems0, sems1) = rest
    vals = [vals0, vals1]
    flags = [flags0, flags1]
    semg = [semg0, semg1]
    sems = [sems0, sems1]

    c = lax.axis_index("c")
    s = lax.axis_index("s")
    cn = c * N
    n0 = s * RPT

    def load_idx(p, g):
        off = s * (DPT * K) + g * GR
        for j in range(NJ):
            pltpu.sync_copy(det.at[pl.ds(off + j * 128, 128)], idxr[p][j])
        for j in range(NJ):
            for t in range(8):
                sl = pl.ds(t * 16, 16)
                idxg[p][j][sl] = idxr[p][j][sl] + cn

    def issue_gather(p):
        for j in range(NJ):
            pltpu.async_copy(x2.at[idxg[p][j]],
                             vals[p].at[pl.ds(j * 128, 128)], semg[p])

    def wait_gather(p):
        for j in range(NJ):
            pltpu.make_async_copy(x2.at[idxg[p][j]],
                                  vals[p].at[pl.ds(j * 128, 128)],
                                  semg[p]).wait()

    def issue_scatter(p):
        for j in range(NJ):
            for t in range(8):
                sl = pl.ds(t * 16, 16)
                idxs[p][j][sl] = idxr[p][j][sl]
        for j in range(NJ):
            pltpu.async_copy(flags[p].at[pl.ds(j * 128, 128)],
                             stat.at[idxs[p][j]], sems[p], add=True)

    def wait_scatter(p):
        for j in range(NJ):
            pltpu.make_async_copy(flags[p].at[pl.ds(j * 128, 128)],
                                  stat.at[idxs[p][j]], sems[p]).wait()

    def compute(p):
        vp = vals[p]
        fp = flags[p]

        def chunk_body(q, cc):
            cb = q * (CH * K)
            for g in range(CH):
                r0 = g * K
                packed = []
                for h in (0, 1):
                    sl = pl.ds(h * 16, 16)
                    v = [vp[cb + r0 + k, sl] for k in range(K)]
                    m = v[0]
                    for k in range(1, K):
                        m = jnp.maximum(m, v[k])
                    eq = v[0] == m
                    wins = [eq]
                    seen = eq
                    for k in range(1, K):
                        eq = v[k] == m
                        wins.append(eq & ~seen)
                        seen = seen | eq
                    packed.append([jnp.where(w, 0, 1).astype(jnp.int32)
                                   for w in wins])
                for k in range(K):
                    both = packed[0][k] | lax.shift_left(packed[1][k], 16)
                    fp[cb + r0 + k, :] = plsc.bitcast(both, jnp.int16)
            return cc

        lax.fori_loop(0, GD // CH, chunk_body, 0)

    load_idx(0, 0)
    issue_gather(0)

    def zrow(r, cc):
        flags1[r, :] = jnp.zeros((BH,), jnp.int16)
        return cc

    lax.fori_loop(0, GR, zrow, 0)
    for j in range(RPT // GR):
        pltpu.sync_copy(flags1, stat.at[pl.ds(n0 + j * GR, GR)])
    plsc.subcore_barrier()

    def pair(i, cc):
        ga = 2 * i
        load_idx(1, ga + 1)
        issue_gather(1)
        wait_gather(0)

        @pl.when(i > 0)
        def _():
            wait_scatter(0)

        compute(0)
        issue_scatter(0)

        @pl.when(i < NG // 2 - 1)
        def _():
            load_idx(0, ga + 2)
            issue_gather(0)

        wait_gather(1)

        @pl.when(i > 0)
        def _():
            wait_scatter(1)

        compute(1)
        issue_scatter(1)
        return cc

    lax.fori_loop(0, NG // 2, pair, 0)
    wait_scatter(0)
    wait_scatter(1)
    plsc.subcore_barrier()

    def o_blk(j, cc):
        r0 = n0 + j * GR
        pltpu.sync_copy(stat.at[pl.ds(r0, GR)], flags0)
        pltpu.sync_copy(x2.at[pl.ds(cn + r0, GR)], vals0)

        def o_sub(q, cc2):
            for r in range(RB):
                row = q * RB + r
                w = plsc.bitcast(flags0[row, :], jnp.int32)
                a = w & 0xFFFF
                bm = lax.shift_right_logical(w, 16)
                x0 = vals0[row, pl.ds(0, 16)]
                x1 = vals0[row, pl.ds(16, 16)]
                vals0[row, pl.ds(0, 16)] = jnp.where(a == 0, x0, 0.0)
                vals0[row, pl.ds(16, 16)] = jnp.where(bm == 0, x1, 0.0)
            return cc2

        lax.fori_loop(0, GR // RB, o_sub, 0)
        pltpu.sync_copy(vals0, out.at[pl.ds(cn + r0, GR)])
        return cc

    lax.fori_loop(0, RPT // GR, o_blk, 0)


_sc_call = functools.partial(
    pl.kernel,
    out_type=jax.ShapeDtypeStruct((NC * N, BH), jnp.float32),
    mesh=plsc.VectorSubcoreMesh(core_axis_name="c", subcore_axis_name="s"),
    compiler_params=pltpu.CompilerParams(
        needs_layout_passes=False, use_tc_tiling_on_sc=False),
    scratch_types=(
        [pltpu.VMEM_SHARED((N, BH), jnp.int16)]
        + [pltpu.VMEM((128,), jnp.int32)] * (3 * 2 * NJ)
        + [pltpu.VMEM((GR, BH), jnp.float32),
           pltpu.VMEM((GR, BH), jnp.float32),
           pltpu.VMEM((GR, BH), jnp.int16),
           pltpu.VMEM((GR, BH), jnp.int16),
           pltpu.SemaphoreType.DMA,
           pltpu.SemaphoreType.DMA,
           pltpu.SemaphoreType.DMA,
           pltpu.SemaphoreType.DMA]
    ),
)(_body)


@jax.jit
def kernel(x, detectors):
    x2 = x.reshape(NC, BH, N).transpose(0, 2, 1).reshape(NC * N, BH)
    det = detectors.reshape(-1)
    out2 = _sc_call(x2, det)
    return out2.reshape(NC, N, BH).transpose(0, 2, 1).reshape(B, N)
